# baseline (speedup 1.0000x reference)
import functools
import numpy as np
import jax
import jax.numpy as jnp
from jax import lax
from jax.experimental import pallas as pl
from jax.experimental.pallas import tpu as pltpu
from jax.experimental.pallas import tpu_sc as plsc

_B, _L, _V, _D = 1024, 200, 100000, 64

_NC, _NS = 2, 16
_W = _NC * _NS
_RPW = _B * _L // _W
_G = 128
_NG = 2
_C = _G * _NG
_NCHUNK = _RPW // _C
_IR = _RPW // _G


def _pos_encoding_host():
    pos = np.arange(_L)[:, np.newaxis]
    i = np.arange(_D)[np.newaxis, :]
    angle_rates = 1.0 / np.power(10000, 2 * (i // 2) / np.float32(_D))
    angles = pos * angle_rates
    angles[:, 0::2] = np.sin(angles[:, 0::2])
    angles[:, 1::2] = np.cos(angles[:, 1::2])
    return np.asarray(angles, dtype=np.float32)


_MESH = plsc.VectorSubcoreMesh(core_axis_name="c", subcore_axis_name="s")


@functools.partial(
    pl.kernel,
    mesh=_MESH,
    out_type=[
        jax.ShapeDtypeStruct((_B * _L, 2 * _D), jnp.float32),
        jax.ShapeDtypeStruct((_B * _L, 2 * _D), jnp.float32),
    ],
    scratch_types=[
        pltpu.VMEM((_IR, _G), jnp.int32),
        pltpu.VMEM((2, _C, _D), jnp.float32),
        pltpu.VMEM((2, _C, 2 * _D), jnp.float32),
        pltpu.VMEM((_L, _D), jnp.float32),
        pltpu.SemaphoreType.DMA((2,)),
        pltpu.SemaphoreType.DMA((2,)),
    ],
    compiler_params=pltpu.CompilerParams(use_tc_tiling_on_sc=False),
)
def _embed_pe_kernel(pe_hbm, gidx_hbm, eidx_hbm, gtab_hbm, etab_hbm,
                     gout_hbm, eout_hbm, idx_v, gath_v, rows_v, pe_v,
                     gsem, osem):
    wid = lax.axis_index("s") * _NC + lax.axis_index("c")
    base = wid * _RPW
    pltpu.sync_copy(pe_hbm, pe_v)

    def fire_gathers(tab_hbm, c, b):
        for g in range(_NG):
            pltpu.async_copy(
                tab_hbm.at[idx_v.at[c * _NG + g]],
                gath_v.at[b, pl.ds(g * _G, _G)],
                gsem.at[b],
            )

    def wait_gathers(tab_hbm, b):
        pltpu.make_async_copy(
            tab_hbm.at[idx_v.at[0]], gath_v.at[b], gsem.at[b]).wait()

    def wait_store(out_hbm, b):
        pltpu.make_async_copy(
            rows_v.at[b], out_hbm.at[pl.ds(0, _C)], osem.at[b]).wait()

    def one_table(idx_hbm, tab_hbm, out_hbm):
        pltpu.sync_copy(idx_hbm.at[pl.ds(wid * _IR, _IR)], idx_v)
        fire_gathers(tab_hbm, 0, 0)

        def chunk_iter(c, carry):
            b = lax.rem(c, 2)

            @pl.when(c + 1 < _NCHUNK)
            def _():
                fire_gathers(tab_hbm, c + 1, 1 - b)

            wait_gathers(tab_hbm, b)

            @pl.when(c >= 2)
            def _():
                wait_store(out_hbm, b)

            pe0 = lax.rem(c * _C, _L)

            def add_body(i, rp):
                for q in range(_D // 16):
                    sl = pl.ds(q * 16, 16)
                    rows_v[b, i, sl] = gath_v[b, i, sl] + pe_v[rp, sl]
                return lax.select(rp == _L - 1, 0, rp + 1)

            lax.fori_loop(0, _C, add_body, pe0, unroll=2)
            off = base + c * _C
            pltpu.async_copy(
                rows_v.at[b], out_hbm.at[pl.ds(off, _C)], osem.at[b])
            return carry

        lax.fori_loop(0, _NCHUNK, chunk_iter, 0)
        wait_store(out_hbm, (_NCHUNK - 1) % 2)
        wait_store(out_hbm, (_NCHUNK - 2) % 2)

    one_table(gidx_hbm, gtab_hbm, gout_hbm)
    one_table(eidx_hbm, etab_hbm, eout_hbm)


def kernel(g_text, e_text, g_table, e_table):
    pe = jnp.asarray(_pos_encoding_host())
    g_idx = g_text.reshape(_B * _L // _G, _G)
    e_idx = e_text.reshape(_B * _L // _G, _G)
    g_out, e_out = _embed_pe_kernel(pe, g_idx, e_idx, g_table, e_table)
    g_out = g_out.reshape(_B, _L, 2 * _D)[:, :, :_D]
    e_out = e_out.reshape(_B, _L, 2 * _D)[:, :, :_D]
    return (g_out, e_out)


# speedup vs baseline: 1.2329x; 1.2329x over previous
import functools
import numpy as np
import jax
import jax.numpy as jnp
from jax import lax
from jax.experimental import pallas as pl
from jax.experimental.pallas import tpu as pltpu
from jax.experimental.pallas import tpu_sc as plsc

_B, _L, _V, _D = 1024, 200, 100000, 64

_NC, _NS = 2, 16
_W = _NC * _NS
_RPW = _B * _L // _W
_G = 128
_NG = 5
_C = _G * _NG
_NCHUNK = _RPW // _C


def _pos_encoding_host():
    pos = np.arange(_L)[:, np.newaxis]
    i = np.arange(_D)[np.newaxis, :]
    angle_rates = 1.0 / np.power(10000, 2 * (i // 2) / np.float32(_D))
    angles = pos * angle_rates
    angles[:, 0::2] = np.sin(angles[:, 0::2])
    angles[:, 1::2] = np.cos(angles[:, 1::2])
    return np.asarray(angles, dtype=np.float32)


def _chunk_pe_segments(c):
    segs = []
    r = 0
    while r < _C:
        pe0 = (c * _C + r) % _L
        n = min(_L - pe0, _C - r)
        segs.append((r, pe0, n))
        r += n
    return segs


_MESH = plsc.VectorSubcoreMesh(core_axis_name="c", subcore_axis_name="s")


def _make_embed_pe_kernel(name):
    @functools.partial(
        pl.kernel,
        mesh=_MESH,
        out_type=jax.ShapeDtypeStruct((_B * _L, _D), jnp.float32),
        scratch_types=[
            pltpu.VMEM((_RPW,), jnp.int32),
            pltpu.VMEM((2, _C, _D), jnp.float32),
            pltpu.VMEM((_L, _D), jnp.float32),
            pltpu.SemaphoreType.DMA,
            pltpu.SemaphoreType.DMA,
            pltpu.SemaphoreType.DMA,
            pltpu.SemaphoreType.DMA,
        ],
        compiler_params=pltpu.CompilerParams(use_tc_tiling_on_sc=False),
        name=name,
    )
    def _embed_pe_kernel(pe_hbm, idx_hbm, tab_hbm, out_hbm, idx_v, rows_v,
                         pe_v, gsem0, gsem1, osem0, osem1):
        wid = lax.axis_index("s") * _NC + lax.axis_index("c")
        base = wid * _RPW
        pltpu.sync_copy(pe_hbm, pe_v)
        gsems = (gsem0, gsem1)
        osems = (osem0, osem1)
        pending_out = [None, None]

        def fire(c):
            b = c % 2
            if pending_out[b] is not None:
                pending_out[b].wait()
                pending_out[b] = None
            return [
                pltpu.async_copy(
                    tab_hbm.at[idx_v.at[pl.ds(c * _C + g * _G, _G)]],
                    rows_v.at[b, pl.ds(g * _G, _G)],
                    gsems[b],
                )
                for g in range(_NG)
            ]

        pltpu.sync_copy(idx_hbm.at[pl.ds(base, _RPW)], idx_v)
        pend = fire(0)
        for c in range(_NCHUNK):
            b = c % 2
            if c + 1 < _NCHUNK:
                nxt = fire(c + 1)
            else:
                nxt = None
            for cp in pend:
                cp.wait()
            for row0, pe0, n in _chunk_pe_segments(c):
                def add_body(i, carry, b=b, row0=row0, pe0=pe0):
                    for q in range(_D // 16):
                        sl = pl.ds(q * 16, 16)
                        plsc.addupdate(
                            rows_v.at[b, row0 + i, sl], pe_v[pe0 + i, sl])
                    return carry
                lax.fori_loop(0, n, add_body, 0, unroll=2)
            off = base + c * _C
            pending_out[b] = pltpu.async_copy(
                rows_v.at[b], out_hbm.at[pl.ds(off, _C)], osems[b])
            pend = nxt
        for b in range(2):
            if pending_out[b] is not None:
                pending_out[b].wait()

    return _embed_pe_kernel


_embed_g = _make_embed_pe_kernel("embed_pe_g")
_embed_e = _make_embed_pe_kernel("embed_pe_e")


def kernel(g_text, e_text, g_table, e_table):
    pe = jnp.asarray(_pos_encoding_host())
    g_out = _embed_g(pe, g_text.reshape(_B * _L), g_table)
    e_out = _embed_e(pe, e_text.reshape(_B * _L), e_table)
    return (g_out.reshape(_B, _L, _D), e_out.reshape(_B, _L, _D))
